# baseline (speedup 1.0000x reference)
import functools

import jax
import jax.numpy as jnp
from jax import lax
from jax.experimental import pallas as pl
from jax.experimental.pallas import tpu as pltpu
from jax.experimental.pallas import tpu_sc as plsc

BATCH = 16384
EMBED = 64
HIDDEN = 256
VOCAB = 1000000

CB = 4096
RGRID = -(-VOCAB // CB)
VROWS = RGRID * (CB // 2)

NC, NS = 2, 16
NW = NC * NS
BPW = BATCH // NW
ICHUNK = 128
NCHUNK = BPW // ICHUNK
WROWS = 256

_SC_MESH = plsc.VectorSubcoreMesh(
    core_axis_name="c", subcore_axis_name="s", num_cores=NC, num_subcores=NS
)


def _relayout_body(u_ref, i_ref, uo_ref, io_ref):
    u = u_ref[...]
    i = i_ref[...]
    uo_ref[:, :EMBED] = u[:, :CB // 2].T
    uo_ref[:, EMBED:] = u[:, CB // 2:].T
    io_ref[:, :EMBED] = i[:, :CB // 2].T
    io_ref[:, EMBED:] = i[:, CB // 2:].T


@jax.jit
def _tc_relayout(utT, itT):
    return pl.pallas_call(
        _relayout_body,
        grid=(RGRID,),
        in_specs=[
            pl.BlockSpec((EMBED, CB), lambda i: (0, i)),
            pl.BlockSpec((EMBED, CB), lambda i: (0, i)),
        ],
        out_specs=[
            pl.BlockSpec((CB // 2, 2 * EMBED), lambda i: (i, 0)),
            pl.BlockSpec((CB // 2, 2 * EMBED), lambda i: (i, 0)),
        ],
        out_shape=[
            jax.ShapeDtypeStruct((VROWS, 2 * EMBED), jnp.float32),
            jax.ShapeDtypeStruct((VROWS, 2 * EMBED), jnp.float32),
        ],
        compiler_params=pltpu.CompilerParams(
            dimension_semantics=("arbitrary",),
        ),
    )(utT, itT)


@functools.partial(
    pl.kernel,
    out_type=(
        jax.ShapeDtypeStruct((BATCH, 2 * EMBED), jnp.float32),
        jax.ShapeDtypeStruct((BATCH, 2 * EMBED), jnp.float32),
    ),
    mesh=_SC_MESH,
    scratch_types=[
        pltpu.VMEM((NCHUNK, ICHUNK), jnp.int32),
        pltpu.VMEM((NCHUNK, ICHUNK), jnp.int32),
        pltpu.VMEM((WROWS, 2 * EMBED), jnp.float32),
        pltpu.VMEM((WROWS, 2 * EMBED), jnp.float32),
        pltpu.SemaphoreType.DMA,
        pltpu.SemaphoreType.DMA,
    ],
)
def _sc_gather(u_tab, i_tab, u_idx, i_idx, u_out, i_out,
               uidx_v, iidx_v, buf0, buf1, gsem, wbsem):
    wid = lax.axis_index("s") * NC + lax.axis_index("c")
    base = wid * BPW
    pltpu.sync_copy(u_idx.at[pl.ds(wid * NCHUNK, NCHUNK)], uidx_v)
    pltpu.sync_copy(i_idx.at[pl.ds(wid * NCHUNK, NCHUNK)], iidx_v)

    phases = [(u_tab, u_out, uidx_v, 0), (u_tab, u_out, uidx_v, 1),
              (i_tab, i_out, iidx_v, 0), (i_tab, i_out, iidx_v, 1)]
    for p, (tab, out, idxv, w) in enumerate(phases):
        buf = buf0 if p % 2 == 0 else buf1
        if p >= 2:
            pltpu.make_async_copy(
                buf, out.at[pl.ds(base + w * WROWS, WROWS)], wbsem
            ).wait()
        g0 = pltpu.async_copy(
            tab.at[idxv.at[2 * w]], buf.at[pl.ds(0, ICHUNK)], gsem)
        g1 = pltpu.async_copy(
            tab.at[idxv.at[2 * w + 1]], buf.at[pl.ds(ICHUNK, ICHUNK)], gsem)
        g0.wait()
        g1.wait()
        pltpu.make_async_copy(
            buf, out.at[pl.ds(base + w * WROWS, WROWS)], wbsem
        ).start()
    for buf, (tab, out, idxv, w) in zip((buf0, buf1), phases[2:]):
        pltpu.make_async_copy(
            buf, out.at[pl.ds(base + w * WROWS, WROWS)], wbsem
        ).wait()


BLK = 1024


def _mlp_body(xu_ref, xi_ref, pu_ref, pi_ref,
              w1u_ref, w1i_ref, b1_ref, w2_ref, b2_ref, o_ref):
    xu = xu_ref[...]
    xi = xi_ref[...]
    pu = pu_ref[...]
    pi = pi_ref[...]
    u = xu[:, :EMBED] + pu * (xu[:, EMBED:] - xu[:, :EMBED])
    it = xi[:, :EMBED] + pi * (xi[:, EMBED:] - xi[:, :EMBED])
    h = (jnp.dot(u, w1u_ref[...], preferred_element_type=jnp.float32)
         + jnp.dot(it, w1i_ref[...], preferred_element_type=jnp.float32)
         + b1_ref[...])
    h = jnp.maximum(h, 0.0)
    o_ref[...] = (jnp.dot(h, w2_ref[...], preferred_element_type=jnp.float32)
                  + b2_ref[...])


@jax.jit
def _tc_mlp(xu, xi, pu, pi, w1u, w1i, b1, w2, b2):
    return pl.pallas_call(
        _mlp_body,
        grid=(BATCH // BLK,),
        in_specs=[
            pl.BlockSpec((BLK, 2 * EMBED), lambda i: (i, 0)),
            pl.BlockSpec((BLK, 2 * EMBED), lambda i: (i, 0)),
            pl.BlockSpec((BLK, 1), lambda i: (i, 0)),
            pl.BlockSpec((BLK, 1), lambda i: (i, 0)),
            pl.BlockSpec((EMBED, HIDDEN), lambda i: (0, 0)),
            pl.BlockSpec((EMBED, HIDDEN), lambda i: (0, 0)),
            pl.BlockSpec((1, HIDDEN), lambda i: (0, 0)),
            pl.BlockSpec((HIDDEN, 1), lambda i: (0, 0)),
            pl.BlockSpec((1, 1), lambda i: (0, 0)),
        ],
        out_specs=pl.BlockSpec((BLK, 1), lambda i: (i, 0)),
        out_shape=jax.ShapeDtypeStruct((BATCH, 1), jnp.float32),
    )(xu, xi, pu, pi, w1u, w1i, b1, w2, b2)


def kernel(user_vector, item_vector, user_table, item_table, W1, b1, W2, b2):
    u_tab, i_tab = _tc_relayout(user_table.T, item_table.T)
    u_idx2 = (((user_vector >> 12) << 11) | (user_vector & 2047)
              ).reshape(NW * NCHUNK, ICHUNK)
    i_idx2 = (((item_vector >> 12) << 11) | (item_vector & 2047)
              ).reshape(NW * NCHUNK, ICHUNK)
    pu = ((user_vector >> 11) & 1).astype(jnp.float32).reshape(BATCH, 1)
    pi = ((item_vector >> 11) & 1).astype(jnp.float32).reshape(BATCH, 1)
    xu, xi = _sc_gather(u_tab, i_tab, u_idx2, i_idx2)
    return _tc_mlp(xu, xi, pu, pi, W1[:EMBED], W1[EMBED:],
                   b1.reshape(1, HIDDEN), W2, b2.reshape(1, 1))


# speedup vs baseline: 1.4547x; 1.4547x over previous
import functools

import jax
import jax.numpy as jnp
from jax import lax
from jax.experimental import pallas as pl
from jax.experimental.pallas import tpu as pltpu
from jax.experimental.pallas import tpu_sc as plsc

BATCH = 16384
EMBED = 64
HIDDEN = 256
VOCAB = 1000000

CB = 8192
RGRID = -(-VOCAB // CB)
VROWS = RGRID * (CB // 4)

NC, NS = 2, 16
NW = NC * NS
BPW = BATCH // NW
ICHUNK = 128
NCHUNK = BPW // ICHUNK
WROWS = 256

_SC_MESH = plsc.VectorSubcoreMesh(
    core_axis_name="c", subcore_axis_name="s", num_cores=NC, num_subcores=NS
)


def _pack(a, b):
    ua = lax.bitcast_convert_type(a.astype(jnp.bfloat16), jnp.uint16
                                  ).astype(jnp.uint32)
    ub = lax.bitcast_convert_type(b.astype(jnp.bfloat16), jnp.uint16
                                  ).astype(jnp.uint32)
    return lax.bitcast_convert_type((ua << 16) | ub, jnp.float32)


def _relayout_body(u_ref, i_ref, uo_ref, io_ref):
    u = u_ref[...]
    i = i_ref[...]
    Q = CB // 4
    uo_ref[:, :EMBED] = _pack(u[:, :Q].T, u[:, Q:2 * Q].T)
    uo_ref[:, EMBED:] = _pack(u[:, 2 * Q:3 * Q].T, u[:, 3 * Q:].T)
    io_ref[:, :EMBED] = _pack(i[:, :Q].T, i[:, Q:2 * Q].T)
    io_ref[:, EMBED:] = _pack(i[:, 2 * Q:3 * Q].T, i[:, 3 * Q:].T)


@jax.jit
def _tc_relayout(utT, itT):
    return pl.pallas_call(
        _relayout_body,
        grid=(RGRID,),
        in_specs=[
            pl.BlockSpec((EMBED, CB), lambda i: (0, i)),
            pl.BlockSpec((EMBED, CB), lambda i: (0, i)),
        ],
        out_specs=[
            pl.BlockSpec((CB // 4, 2 * EMBED), lambda i: (i, 0)),
            pl.BlockSpec((CB // 4, 2 * EMBED), lambda i: (i, 0)),
        ],
        out_shape=[
            jax.ShapeDtypeStruct((VROWS, 2 * EMBED), jnp.float32),
            jax.ShapeDtypeStruct((VROWS, 2 * EMBED), jnp.float32),
        ],
        compiler_params=pltpu.CompilerParams(
            dimension_semantics=("arbitrary",),
        ),
    )(utT, itT)


@functools.partial(
    pl.kernel,
    out_type=(
        jax.ShapeDtypeStruct((BATCH, 2 * EMBED), jnp.float32),
        jax.ShapeDtypeStruct((BATCH, 2 * EMBED), jnp.float32),
    ),
    mesh=_SC_MESH,
    scratch_types=[
        pltpu.VMEM((NCHUNK, ICHUNK), jnp.int32),
        pltpu.VMEM((NCHUNK, ICHUNK), jnp.int32),
        pltpu.VMEM((WROWS, 2 * EMBED), jnp.float32),
        pltpu.VMEM((WROWS, 2 * EMBED), jnp.float32),
        pltpu.SemaphoreType.DMA,
        pltpu.SemaphoreType.DMA,
    ],
)
def _sc_gather(u_tab, i_tab, u_idx, i_idx, u_out, i_out,
               uidx_v, iidx_v, buf0, buf1, gsem, wbsem):
    wid = lax.axis_index("s") * NC + lax.axis_index("c")
    base = wid * BPW
    pltpu.sync_copy(u_idx.at[pl.ds(wid * NCHUNK, NCHUNK)], uidx_v)
    pltpu.sync_copy(i_idx.at[pl.ds(wid * NCHUNK, NCHUNK)], iidx_v)

    phases = [(u_tab, u_out, uidx_v, 0), (u_tab, u_out, uidx_v, 1),
              (i_tab, i_out, iidx_v, 0), (i_tab, i_out, iidx_v, 1)]
    for p, (tab, out, idxv, w) in enumerate(phases):
        buf = buf0 if p % 2 == 0 else buf1
        if p >= 2:
            pltpu.make_async_copy(
                buf, out.at[pl.ds(base + w * WROWS, WROWS)], wbsem
            ).wait()
        g0 = pltpu.async_copy(
            tab.at[idxv.at[2 * w]], buf.at[pl.ds(0, ICHUNK)], gsem)
        g1 = pltpu.async_copy(
            tab.at[idxv.at[2 * w + 1]], buf.at[pl.ds(ICHUNK, ICHUNK)], gsem)
        g0.wait()
        g1.wait()
        pltpu.make_async_copy(
            buf, out.at[pl.ds(base + w * WROWS, WROWS)], wbsem
        ).start()
    for buf, (tab, out, idxv, w) in zip((buf0, buf1), phases[2:]):
        pltpu.make_async_copy(
            buf, out.at[pl.ds(base + w * WROWS, WROWS)], wbsem
        ).wait()


BLK = 1024


def _unpack(x, p1, p0):
    w = lax.bitcast_convert_type(x, jnp.uint32)
    w64 = jnp.where(p1 > 0.5, w[:, EMBED:], w[:, :EMBED])
    bits = jnp.where(p0 > 0.5, w64 << 16, w64) & jnp.uint32(0xFFFF0000)
    return lax.bitcast_convert_type(bits, jnp.float32)


def _mlp_body(xu_ref, xi_ref, pu1_ref, pi1_ref, pu0_ref, pi0_ref,
              w1u_ref, w1i_ref, b1_ref, w2_ref, b2_ref, o_ref):
    u = _unpack(xu_ref[...], pu1_ref[...], pu0_ref[...])
    it = _unpack(xi_ref[...], pi1_ref[...], pi0_ref[...])
    h = (jnp.dot(u, w1u_ref[...], preferred_element_type=jnp.float32)
         + jnp.dot(it, w1i_ref[...], preferred_element_type=jnp.float32)
         + b1_ref[...])
    h = jnp.maximum(h, 0.0)
    o_ref[...] = (jnp.dot(h, w2_ref[...], preferred_element_type=jnp.float32)
                  + b2_ref[...])


@jax.jit
def _tc_mlp(xu, xi, pu1, pi1, pu0, pi0, w1u, w1i, b1, w2, b2):
    return pl.pallas_call(
        _mlp_body,
        grid=(BATCH // BLK,),
        in_specs=[
            pl.BlockSpec((BLK, 2 * EMBED), lambda i: (i, 0)),
            pl.BlockSpec((BLK, 2 * EMBED), lambda i: (i, 0)),
            pl.BlockSpec((BLK, 1), lambda i: (i, 0)),
            pl.BlockSpec((BLK, 1), lambda i: (i, 0)),
            pl.BlockSpec((BLK, 1), lambda i: (i, 0)),
            pl.BlockSpec((BLK, 1), lambda i: (i, 0)),
            pl.BlockSpec((EMBED, HIDDEN), lambda i: (0, 0)),
            pl.BlockSpec((EMBED, HIDDEN), lambda i: (0, 0)),
            pl.BlockSpec((1, HIDDEN), lambda i: (0, 0)),
            pl.BlockSpec((HIDDEN, 1), lambda i: (0, 0)),
            pl.BlockSpec((1, 1), lambda i: (0, 0)),
        ],
        out_specs=pl.BlockSpec((BLK, 1), lambda i: (i, 0)),
        out_shape=jax.ShapeDtypeStruct((BATCH, 1), jnp.float32),
    )(xu, xi, pu1, pi1, pu0, pi0, w1u, w1i, b1, w2, b2)


def kernel(user_vector, item_vector, user_table, item_table, W1, b1, W2, b2):
    u_tab, i_tab = _tc_relayout(user_table.T, item_table.T)
    u_idx2 = (((user_vector >> 13) << 11) | (user_vector & 2047)
              ).reshape(NW * NCHUNK, ICHUNK)
    i_idx2 = (((item_vector >> 13) << 11) | (item_vector & 2047)
              ).reshape(NW * NCHUNK, ICHUNK)
    pu1 = ((user_vector >> 12) & 1).astype(jnp.float32).reshape(BATCH, 1)
    pi1 = ((item_vector >> 12) & 1).astype(jnp.float32).reshape(BATCH, 1)
    pu0 = ((user_vector >> 11) & 1).astype(jnp.float32).reshape(BATCH, 1)
    pi0 = ((item_vector >> 11) & 1).astype(jnp.float32).reshape(BATCH, 1)
    xu, xi = _sc_gather(u_tab, i_tab, u_idx2, i_idx2)
    return _tc_mlp(xu, xi, pu1, pi1, pu0, pi0, W1[:EMBED], W1[EMBED:],
                   b1.reshape(1, HIDDEN), W2, b2.reshape(1, 1))
